# baseline (speedup 1.0000x reference)
import numpy as np
import jax
import jax.numpy as jnp
from jax.experimental import pallas as pl
from jax.experimental.pallas import tpu as pltpu

_NBINS = 100
_P = 0.95


def _radial_consts(h, w):
    cy, cx = h // 2, w // 2
    yy = jnp.arange(h, dtype=jnp.float32) - cy
    xx = jnp.arange(w, dtype=jnp.float32) - cx
    md = ((h // 2) ** 2 + (w // 2) ** 2) ** 0.5
    yy = yy / (md + 1e-06)
    xx = xx / (md + 1e-06)
    gy, gx = jnp.meshgrid(yy, xx, indexing="ij")
    rr = jnp.sqrt(gx * gx + gy * gy)
    r_max = jnp.max(rr)
    bin_idx = jnp.floor(rr / r_max * _NBINS).astype(jnp.int32)
    bidx = jnp.where(bin_idx < _NBINS,
                     jnp.clip(bin_idx, 0, _NBINS - 1), 127).astype(jnp.float32)
    edges = jnp.linspace(0.0, r_max, _NBINS + 1)
    radii = (edges[:-1] + edges[1:]) * 0.5
    radii_pad = jnp.concatenate(
        [radii, jnp.full((28,), radii[-1], jnp.float32)]).reshape(1, 1, 128)
    return rr, bidx, radii_pad


def _dft_mats(n):
    k = np.arange(n, dtype=np.float64) - n // 2
    y = np.arange(n, dtype=np.float64)
    th = (-2.0 * np.pi / n) * np.outer(k, y)
    return (jnp.asarray(np.cos(th), jnp.float32),
            jnp.asarray(np.sin(th), jnp.float32))


def _split_hi_lo(a):
    hi = a.astype(jnp.bfloat16)
    lo = (a - hi.astype(jnp.float32)).astype(jnp.bfloat16)
    return hi, lo


def _make_body(C, H, W):
    def dotg(a, b, dims):
        return jax.lax.dot_general(a, b, (dims, ((), ())),
                                   preferred_element_type=jnp.float32)

    def dcat1(v):
        vh, vl = _split_hi_lo(v)
        return jnp.concatenate([vh, vh, vl], axis=1)

    def dcat0(v):
        vh, vl = _split_hi_lo(v)
        return jnp.concatenate([vh, vl, vh], axis=0)

    def body(x_ref, rr_ref, bidx_ref, radii_ref,
             f1r_ref, f1i_ref, f2r_ref, f2i_ref, f2s_ref,
             i1r_ref, i1i_ref, i1d_ref, i2r_ref, i2i_ref,
             ih_ref, il_ref, d0_ref, ml_ref, mh_ref,
             fr_scr, fi_scr, pw_scr):
        s = pl.program_id(1)
        X = x_ref[0, 0]

        @pl.when(s < C)
        def _fwd():
            Xc = dcat1(X)
            Yr = dotg(Xc, f1r_ref[...], ((1,), (1,)))
            Yi = dotg(Xc, f1i_ref[...], ((1,), (1,)))
            P1 = dotg(f2r_ref[...], dcat0(Yr), ((1,), (0,)))
            P2 = dotg(f2i_ref[...], dcat0(Yi), ((1,), (0,)))
            P3 = dotg(f2s_ref[...], dcat0(Yr + Yi), ((1,), (0,)))
            Fr = P1 - P2
            Fi = P3 - P1 - P2
            fr_scr[pl.ds(s, 1)] = Fr[None]
            fi_scr[pl.ds(s, 1)] = Fi[None]
            p = Fr * Fr + Fi * Fi

            @pl.when(s == 0)
            def _():
                pw_scr[...] = p

            @pl.when(s > 0)
            def _():
                pw_scr[...] = pw_scr[...] + p

        @pl.when(s == C - 1)
        def _d0():
            pw = pw_scr[...]
            total = jnp.maximum(jnp.sum(pw), 1e-12)
            bidx = bidx_ref[...]
            thr = _P * total

            def loop(_, carry):
                lo, hi = carry
                mid = (lo + hi) // 2
                c = jnp.sum(jnp.where(bidx <= mid.astype(jnp.float32),
                                      pw, 0.0))
                take = c >= thr
                return (jnp.where(take, lo, mid), jnp.where(take, mid, hi))

            lo, hi = jax.lax.fori_loop(
                0, 7, loop, (jnp.int32(-1), jnp.int32(_NBINS)))
            kf = jnp.minimum(hi, _NBINS - 1).astype(jnp.float32)
            lane = jax.lax.broadcasted_iota(
                jnp.int32, (1, 128), 1).astype(jnp.float32)
            d0 = jnp.sum(jnp.where(lane == kf, radii_ref[0], 0.0))
            d0_ref[...] = jnp.broadcast_to(d0, (1, 1, 1))

        @pl.when(s >= C)
        def _inv():
            cc = s - C
            Fr = fr_scr[pl.ds(cc, 1)][0]
            Fi = fi_scr[pl.ds(cc, 1)][0]
            d0 = d0_ref[...][0, 0, 0]
            m = (rr_ref[...] <= d0).astype(jnp.float32)
            Gr = Fr * m
            Gi = Fi * m
            P1 = dotg(i1r_ref[...], dcat0(Gr), ((0,), (0,)))
            M2 = dotg(i1i_ref[...], dcat0(Gi), ((0,), (0,)))
            P3 = dotg(i1d_ref[...], dcat0(Gr + Gi), ((0,), (0,)))
            Zr = P1 + M2
            Zi = P3 - P1 + M2
            IL = (dotg(dcat1(Zr), i2r_ref[...], ((1,), (0,))) +
                  dotg(dcat1(Zi), i2i_ref[...], ((1,), (0,)))) * (1.0 / (H * W))
            il_ref[0, 0] = IL
            ih_ref[0, 0] = X - IL
            ml_ref[0, 0] = m
            mh_ref[0, 0] = 1.0 - m

    return body


def kernel(x):
    B, C, H, W = x.shape
    rr, bidx, radii_pad = _radial_consts(H, W)
    Dr, Di = _dft_mats(H)

    Rrh, Rrl = _split_hi_lo(Dr)
    Rih, Ril = _split_hi_lo(Di)
    Rsh, Rsl = _split_hi_lo(Dr + Di)
    Rdh, Rdl = _split_hi_lo(Dr - Di)
    cat = jnp.concatenate
    mats = [
        cat([Rrh, Rrl, Rrh], axis=1),
        cat([Rih, Ril, Rih], axis=1),
        cat([Rrh, Rrh, Rrl], axis=1),
        cat([Rih, Rih, Ril], axis=1),
        cat([Rsh, Rsh, Rsl], axis=1),
        cat([Rrh, Rrh, Rrl], axis=0),
        cat([Rih, Rih, Ril], axis=0),
        cat([Rdh, Rdh, Rdl], axis=0),
        cat([Rrh, Rrl, Rrh], axis=0),
        cat([Rih, Ril, Rih], axis=0),
    ]

    body = _make_body(C, H, W)

    def xmap(b, s):
        return (b, jnp.where(s < C, s, s - C), 0, 0)

    def omap(b, s):
        return (b, jnp.where(s < C, 0, s - C), 0, 0)

    wide = pl.BlockSpec((H, 3 * H), lambda b, s: (0, 0))
    tall = pl.BlockSpec((3 * H, H), lambda b, s: (0, 0))

    outs = pl.pallas_call(
        body,
        grid=(B, 2 * C),
        in_specs=[
            pl.BlockSpec((1, 1, H, W), xmap),
            pl.BlockSpec((H, W), lambda b, s: (0, 0)),
            pl.BlockSpec((H, W), lambda b, s: (0, 0)),
            pl.BlockSpec((1, 1, 128), lambda b, s: (0, 0, 0)),
        ] + [wide] * 5 + [tall] * 5,
        out_specs=[
            pl.BlockSpec((1, 1, H, W), omap),
            pl.BlockSpec((1, 1, H, W), omap),
            pl.BlockSpec((1, 1, 1), lambda b, s: (b, 0, 0)),
            pl.BlockSpec((1, 1, H, W), lambda b, s: (b, 0, 0, 0)),
            pl.BlockSpec((1, 1, H, W), lambda b, s: (b, 0, 0, 0)),
        ],
        out_shape=[
            jax.ShapeDtypeStruct((B, C, H, W), jnp.float32),
            jax.ShapeDtypeStruct((B, C, H, W), jnp.float32),
            jax.ShapeDtypeStruct((B, 1, 1), jnp.float32),
            jax.ShapeDtypeStruct((B, 1, H, W), jnp.float32),
            jax.ShapeDtypeStruct((B, 1, H, W), jnp.float32),
        ],
        scratch_shapes=[
            pltpu.VMEM((C, H, W), jnp.float32),
            pltpu.VMEM((C, H, W), jnp.float32),
            pltpu.VMEM((H, W), jnp.float32),
        ],
        compiler_params=pltpu.CompilerParams(
            dimension_semantics=("parallel", "arbitrary")),
    )(x, rr, bidx, radii_pad, *mats)

    i_high, i_low, d0, mask_low, mask_high = outs
    return i_high, i_low, d0.reshape(B), mask_low, mask_high


# speedup vs baseline: 1.3237x; 1.3237x over previous
import numpy as np
import jax
import jax.numpy as jnp
from jax.experimental import pallas as pl
from jax.experimental.pallas import tpu as pltpu

_NBINS = 100
_P = 0.95


def _radial_consts(h, w):
    cy, cx = h // 2, w // 2
    yy = jnp.arange(h, dtype=jnp.float32) - cy
    xx = jnp.arange(w, dtype=jnp.float32) - cx
    md = ((h // 2) ** 2 + (w // 2) ** 2) ** 0.5
    yy = yy / (md + 1e-06)
    xx = xx / (md + 1e-06)
    gy, gx = jnp.meshgrid(yy, xx, indexing="ij")
    rr = jnp.sqrt(gx * gx + gy * gy)
    r_max = jnp.max(rr)
    bin_idx = jnp.floor(rr / r_max * _NBINS).astype(jnp.int32)
    bidx = jnp.where(bin_idx < _NBINS,
                     jnp.clip(bin_idx, 0, _NBINS - 1), 127).astype(jnp.float32)
    edges = jnp.linspace(0.0, r_max, _NBINS + 1)
    radii = (edges[:-1] + edges[1:]) * 0.5
    radii_pad = jnp.concatenate(
        [radii, jnp.full((28,), radii[-1], jnp.float32)]).reshape(1, 1, 128)
    return rr, bidx, radii_pad


def _dft_mats(n):
    k = np.arange(n, dtype=np.float64) - n // 2
    y = np.arange(n, dtype=np.float64)
    th = (-2.0 * np.pi / n) * np.outer(k, y)
    return (jnp.asarray(np.cos(th), jnp.float32),
            jnp.asarray(np.sin(th), jnp.float32))


def _split_hi_lo(a):
    hi = a.astype(jnp.bfloat16)
    lo = (a - hi.astype(jnp.float32)).astype(jnp.bfloat16)
    return hi, lo


def _make_body(C, H, W):
    def dotg(a, b, dims):
        return jax.lax.dot_general(a, b, (dims, ((), ())),
                                   preferred_element_type=jnp.float32)

    def dcat1(v):
        vh, vl = _split_hi_lo(v)
        return jnp.concatenate([vh, vh, vl], axis=1)

    def dcat0(v):
        vh, vl = _split_hi_lo(v)
        return jnp.concatenate([vh, vl, vh], axis=0)

    def body(x_ref, rr_ref, bidx_ref, radii_ref,
             f1r_ref, f1i_ref, f1s_ref, f2r_ref, f2i_ref, f2s_ref,
             i1r_ref, i1i_ref, i1d_ref, i2r_ref, i2i_ref, i2d_ref,
             ih_ref, il_ref, d0_ref, ml_ref, mh_ref,
             fr_scr, fi_scr, pw_scr):
        s = pl.program_id(1)

        @pl.when(s == 0)
        def _fwd_pair():
            X0 = x_ref[0, 0]
            X1 = x_ref[0, 1]
            P1 = dotg(dcat1(X0), f1r_ref[...], ((1,), (1,)))
            P2 = dotg(dcat1(X1), f1i_ref[...], ((1,), (1,)))
            P3 = dotg(dcat1(X0 + X1), f1s_ref[...], ((1,), (1,)))
            Yr = P1 - P2
            Yi = P3 - P1 - P2
            Q1 = dotg(f2r_ref[...], dcat0(Yr), ((1,), (0,)))
            Q2 = dotg(f2i_ref[...], dcat0(Yi), ((1,), (0,)))
            Q3 = dotg(f2s_ref[...], dcat0(Yr + Yi), ((1,), (0,)))
            Fzr = Q1 - Q2
            Fzi = Q3 - Q1 - Q2
            fr_scr[0] = Fzr
            fi_scr[0] = Fzi
            pw_scr[...] = Fzr * Fzr + Fzi * Fzi

        @pl.when(s == 1)
        def _fwd_single():
            X = x_ref[0, 0]
            Yr = dotg(dcat1(X), f1r_ref[...], ((1,), (1,)))
            Yi = dotg(dcat1(X), f1i_ref[...], ((1,), (1,)))
            P1 = dotg(f2r_ref[...], dcat0(Yr), ((1,), (0,)))
            P2 = dotg(f2i_ref[...], dcat0(Yi), ((1,), (0,)))
            P3 = dotg(f2s_ref[...], dcat0(Yr + Yi), ((1,), (0,)))
            Fr = P1 - P2
            Fi = P3 - P1 - P2
            fr_scr[1] = Fr
            fi_scr[1] = Fi
            pw = pw_scr[...] + Fr * Fr + Fi * Fi

            total = jnp.maximum(jnp.sum(pw), 1e-12)
            bidx = bidx_ref[...]
            thr = _P * total

            def loop(_, carry):
                lo, hi = carry
                mid = (lo + hi) // 2
                c = jnp.sum(jnp.where(bidx <= mid.astype(jnp.float32),
                                      pw, 0.0))
                take = c >= thr
                return (jnp.where(take, lo, mid), jnp.where(take, mid, hi))

            lo, hi = jax.lax.fori_loop(
                0, 7, loop, (jnp.int32(-1), jnp.int32(_NBINS)))
            kf = jnp.minimum(hi, _NBINS - 1).astype(jnp.float32)
            lane = jax.lax.broadcasted_iota(
                jnp.int32, (1, 128), 1).astype(jnp.float32)
            d0 = jnp.sum(jnp.where(lane == kf, radii_ref[0], 0.0))
            d0_ref[...] = jnp.broadcast_to(d0, (1, 1, 1))

        @pl.when(s == 2)
        def _inv_pair():
            d0 = d0_ref[...][0, 0, 0]
            m = (rr_ref[...] <= d0).astype(jnp.float32)
            Wr = fr_scr[0] * m
            Wi = fi_scr[0] * m
            P1 = dotg(i1r_ref[...], dcat0(Wr), ((0,), (0,)))
            M2 = dotg(i1i_ref[...], dcat0(Wi), ((0,), (0,)))
            P3 = dotg(i1d_ref[...], dcat0(Wr + Wi), ((0,), (0,)))
            Zr = P1 + M2
            Zi = P3 - P1 + M2
            Zrc = dcat1(Zr)
            Zic = dcat1(Zi)
            Q1 = dotg(Zrc, i2r_ref[...], ((1,), (0,)))
            Q2 = -dotg(Zic, i2i_ref[...], ((1,), (0,)))
            Q3 = dotg(dcat1(Zr + Zi), i2d_ref[...], ((1,), (0,)))
            scale = 1.0 / (H * W)
            IL0 = (Q1 - Q2) * scale
            IL1 = (Q3 - Q1 - Q2) * scale
            il_ref[0, 0] = IL0
            il_ref[0, 1] = IL1
            ih_ref[0, 0] = x_ref[0, 0] - IL0
            ih_ref[0, 1] = x_ref[0, 1] - IL1
            ml_ref[0, 0] = m
            mh_ref[0, 0] = 1.0 - m

        @pl.when(s == 3)
        def _inv_single():
            d0 = d0_ref[...][0, 0, 0]
            m = (rr_ref[...] <= d0).astype(jnp.float32)
            Gr = fr_scr[1] * m
            Gi = fi_scr[1] * m
            P1 = dotg(i1r_ref[...], dcat0(Gr), ((0,), (0,)))
            M2 = dotg(i1i_ref[...], dcat0(Gi), ((0,), (0,)))
            P3 = dotg(i1d_ref[...], dcat0(Gr + Gi), ((0,), (0,)))
            Zr = P1 + M2
            Zi = P3 - P1 + M2
            IL = (dotg(dcat1(Zr), i2r_ref[...], ((1,), (0,))) +
                  dotg(dcat1(Zi), i2i_ref[...], ((1,), (0,)))) * (1.0 / (H * W))
            il_ref[0, 0] = IL
            ih_ref[0, 0] = x_ref[0, 0] - IL

    return body


def kernel(x):
    B, C, H, W = x.shape
    rr, bidx, radii_pad = _radial_consts(H, W)
    Dr, Di = _dft_mats(H)

    Rrh, Rrl = _split_hi_lo(Dr)
    Rih, Ril = _split_hi_lo(Di)
    Rsh, Rsl = _split_hi_lo(Dr + Di)
    Rdh, Rdl = _split_hi_lo(Dr - Di)
    cat = jnp.concatenate
    mats = [
        cat([Rrh, Rrl, Rrh], axis=1),
        cat([Rih, Ril, Rih], axis=1),
        cat([Rsh, Rsl, Rsh], axis=1),
        cat([Rrh, Rrh, Rrl], axis=1),
        cat([Rih, Rih, Ril], axis=1),
        cat([Rsh, Rsh, Rsl], axis=1),
        cat([Rrh, Rrh, Rrl], axis=0),
        cat([Rih, Rih, Ril], axis=0),
        cat([Rdh, Rdh, Rdl], axis=0),
        cat([Rrh, Rrl, Rrh], axis=0),
        cat([Rih, Ril, Rih], axis=0),
        cat([Rdh, Rdl, Rdh], axis=0),
    ]

    body = _make_body(C, H, W)

    def xmap(b, s):
        return (b, jnp.where(s < 2, s, s - 2), 0, 0)

    def omap(b, s):
        return (b, jnp.where(s < 2, 0, s - 2), 0, 0)

    wide = pl.BlockSpec((H, 3 * H), lambda b, s: (0, 0))
    tall = pl.BlockSpec((3 * H, H), lambda b, s: (0, 0))

    outs = pl.pallas_call(
        body,
        grid=(B, 4),
        in_specs=[
            pl.BlockSpec((1, 2, H, W), xmap),
            pl.BlockSpec((H, W), lambda b, s: (0, 0)),
            pl.BlockSpec((H, W), lambda b, s: (0, 0)),
            pl.BlockSpec((1, 1, 128), lambda b, s: (0, 0, 0)),
        ] + [wide] * 6 + [tall] * 6,
        out_specs=[
            pl.BlockSpec((1, 2, H, W), omap),
            pl.BlockSpec((1, 2, H, W), omap),
            pl.BlockSpec((1, 1, 1), lambda b, s: (b, 0, 0)),
            pl.BlockSpec((1, 1, H, W), lambda b, s: (b, 0, 0, 0)),
            pl.BlockSpec((1, 1, H, W), lambda b, s: (b, 0, 0, 0)),
        ],
        out_shape=[
            jax.ShapeDtypeStruct((B, C, H, W), jnp.float32),
            jax.ShapeDtypeStruct((B, C, H, W), jnp.float32),
            jax.ShapeDtypeStruct((B, 1, 1), jnp.float32),
            jax.ShapeDtypeStruct((B, 1, H, W), jnp.float32),
            jax.ShapeDtypeStruct((B, 1, H, W), jnp.float32),
        ],
        scratch_shapes=[
            pltpu.VMEM((2, H, W), jnp.float32),
            pltpu.VMEM((2, H, W), jnp.float32),
            pltpu.VMEM((H, W), jnp.float32),
        ],
        compiler_params=pltpu.CompilerParams(
            dimension_semantics=("arbitrary", "arbitrary")),
    )(x, rr, bidx, radii_pad, *mats)

    i_high, i_low, d0, mask_low, mask_high = outs
    return i_high, i_low, d0.reshape(B), mask_low, mask_high


# speedup vs baseline: 1.7869x; 1.3499x over previous
import numpy as np
import jax
import jax.numpy as jnp
from jax.experimental import pallas as pl
from jax.experimental.pallas import tpu as pltpu

_NBINS = 100
_P = 0.95


def _radial_consts(h, w):
    cy, cx = h // 2, w // 2
    yy = jnp.arange(h, dtype=jnp.float32) - cy
    xx = jnp.arange(w, dtype=jnp.float32) - cx
    md = ((h // 2) ** 2 + (w // 2) ** 2) ** 0.5
    yy = yy / (md + 1e-06)
    xx = xx / (md + 1e-06)
    gy, gx = jnp.meshgrid(yy, xx, indexing="ij")
    rr = jnp.sqrt(gx * gx + gy * gy)
    r_max = jnp.max(rr)
    bin_idx = jnp.floor(rr / r_max * _NBINS).astype(jnp.int32)
    bidx = jnp.where(bin_idx < _NBINS,
                     jnp.clip(bin_idx, 0, _NBINS - 1), 127).astype(jnp.float32)
    edges = jnp.linspace(0.0, r_max, _NBINS + 1)
    radii = (edges[:-1] + edges[1:]) * 0.5
    radii_pad = jnp.concatenate(
        [radii, jnp.full((28,), radii[-1], jnp.float32)]).reshape(1, 1, 128)
    return rr, bidx, radii_pad


def _split_hi_lo(a):
    hi = a.astype(jnp.bfloat16)
    lo = (a - hi.astype(jnp.float32)).astype(jnp.bfloat16)
    return hi, lo


def _make_body(C, H, W):
    Hh = H // 2

    def dotg(a, b, dims):
        return jax.lax.dot_general(a, b, (dims, ((), ())),
                                   preferred_element_type=jnp.float32)

    def dcat1(v):
        vh, vl = _split_hi_lo(v)
        return jnp.concatenate([vh, vh, vl], axis=1)

    def dcat0(v):
        vh, vl = _split_hi_lo(v)
        return jnp.concatenate([vh, vl, vh], axis=0)

    def body(x_ref, rr_ref, rrp_ref, bidxp_ref, radii_ref,
             t1r_ref, t1i_ref, t2r_ref, t2i_ref,
             w2r_ref, w2i_ref, wxr_ref, wxi_ref,
             e1r_ref, e1i_ref, e1s_ref,
             e2r_ref, e2i_ref, e2s_ref,
             j1r_ref, j1i_ref, j1d_ref,
             j2r_ref, j2i_ref, j2d_ref,
             ih_ref, il_ref, d0_ref, ml_ref, mh_ref,
             fr_scr, fi_scr, pw_scr):
        s = pl.program_id(1)

        def t1_complex(Xr, Xi):
            Arr = Xr[:, :Hh] + Xr[:, Hh:]
            Aii = Xi[:, :Hh] + Xi[:, Hh:]
            Brr = Xr[:, :Hh] - Xr[:, Hh:]
            Bii = Xi[:, :Hh] - Xi[:, Hh:]
            t1r = t1r_ref[0]
            t1i = t1i_ref[0]
            Btr = Brr * t1r - Bii * t1i
            Bti = Brr * t1i + Bii * t1r
            P1 = dotg(dcat1(Arr), e1r_ref[...], ((1,), (1,)))
            P2 = dotg(dcat1(Aii), e1i_ref[...], ((1,), (1,)))
            P3 = dotg(dcat1(Arr + Aii), e1s_ref[...], ((1,), (1,)))
            Q1 = dotg(dcat1(Btr), e1r_ref[...], ((1,), (1,)))
            Q2 = dotg(dcat1(Bti), e1i_ref[...], ((1,), (1,)))
            Q3 = dotg(dcat1(Btr + Bti), e1s_ref[...], ((1,), (1,)))
            Yr = jnp.concatenate([P1 - P2, Q1 - Q2], axis=1)
            Yi = jnp.concatenate([P3 - P1 - P2, Q3 - Q1 - Q2], axis=1)
            return Yr, Yi

        def t1_real(X):
            Arr = X[:, :Hh] + X[:, Hh:]
            Brr = X[:, :Hh] - X[:, Hh:]
            t1r = t1r_ref[0]
            t1i = t1i_ref[0]
            Btr = Brr * t1r
            Bti = Brr * t1i
            Ac = dcat1(Arr)
            Yevr = dotg(Ac, e1r_ref[...], ((1,), (1,)))
            Yevi = dotg(Ac, e1i_ref[...], ((1,), (1,)))
            Q1 = dotg(dcat1(Btr), e1r_ref[...], ((1,), (1,)))
            Q2 = dotg(dcat1(Bti), e1i_ref[...], ((1,), (1,)))
            Q3 = dotg(dcat1(Btr + Bti), e1s_ref[...], ((1,), (1,)))
            Yr = jnp.concatenate([Yevr, Q1 - Q2], axis=1)
            Yi = jnp.concatenate([Yevi, Q3 - Q1 - Q2], axis=1)
            return Yr, Yi

        def t2_complex(Yr, Yi):
            A2r = Yr[:Hh, :] + Yr[Hh:, :]
            A2i = Yi[:Hh, :] + Yi[Hh:, :]
            B2r = Yr[:Hh, :] - Yr[Hh:, :]
            B2i = Yi[:Hh, :] - Yi[Hh:, :]
            t2r = t2r_ref[...]
            t2i = t2i_ref[...]
            B2tr = B2r * t2r - B2i * t2i
            B2ti = B2r * t2i + B2i * t2r
            P1 = dotg(e2r_ref[...], dcat0(A2r), ((1,), (0,)))
            P2 = dotg(e2i_ref[...], dcat0(A2i), ((1,), (0,)))
            P3 = dotg(e2s_ref[...], dcat0(A2r + A2i), ((1,), (0,)))
            Q1 = dotg(e2r_ref[...], dcat0(B2tr), ((1,), (0,)))
            Q2 = dotg(e2i_ref[...], dcat0(B2ti), ((1,), (0,)))
            Q3 = dotg(e2s_ref[...], dcat0(B2tr + B2ti), ((1,), (0,)))
            Fr = jnp.concatenate([P1 - P2, Q1 - Q2], axis=0)
            Fi = jnp.concatenate([P3 - P1 - P2, Q3 - Q1 - Q2], axis=0)
            return Fr, Fi

        def i1(Wr, Wi):
            Te_args = (Wr[:Hh, :], Wi[:Hh, :])
            To_args = (Wr[Hh:, :], Wi[Hh:, :])

            def half(ar, ai):
                P1 = dotg(j1r_ref[...], dcat0(ar), ((0,), (0,)))
                M2 = dotg(j1i_ref[...], dcat0(ai), ((0,), (0,)))
                P3 = dotg(j1d_ref[...], dcat0(ar + ai), ((0,), (0,)))
                return P1 + M2, P3 - P1 + M2

            Ter, Tei = half(*Te_args)
            Tor, Toi = half(*To_args)
            TerR = jnp.concatenate([Ter, Ter], axis=0)
            TeiR = jnp.concatenate([Tei, Tei], axis=0)
            TorR = jnp.concatenate([Tor, Tor], axis=0)
            ToiR = jnp.concatenate([Toi, Toi], axis=0)
            w2r = w2r_ref[...]
            w2i = w2i_ref[...]
            Zr = TerR + w2r * TorR - w2i * ToiR
            Zi = TeiR + w2r * ToiR + w2i * TorR
            return Zr, Zi

        def i2_complex(Zr, Zi):
            def half(ar, ai):
                P1 = dotg(dcat1(ar), j2r_ref[...], ((1,), (0,)))
                M2 = dotg(dcat1(ai), j2i_ref[...], ((1,), (0,)))
                P3 = dotg(dcat1(ar + ai), j2d_ref[...], ((1,), (0,)))
                return P1 + M2, P3 - P1 + M2

            Uer, Uei = half(Zr[:, :Hh], Zi[:, :Hh])
            Uor, Uoi = half(Zr[:, Hh:], Zi[:, Hh:])
            UerT = jnp.concatenate([Uer, Uer], axis=1)
            UeiT = jnp.concatenate([Uei, Uei], axis=1)
            UorT = jnp.concatenate([Uor, Uor], axis=1)
            UoiT = jnp.concatenate([Uoi, Uoi], axis=1)
            wxr = wxr_ref[0]
            wxi = wxi_ref[0]
            zr = UerT + wxr * UorT - wxi * UoiT
            zi = UeiT + wxr * UoiT + wxi * UorT
            return zr, zi

        def i2_real(Zr, Zi):
            Uer = (dotg(dcat1(Zr[:, :Hh]), j2r_ref[...], ((1,), (0,))) +
                   dotg(dcat1(Zi[:, :Hh]), j2i_ref[...], ((1,), (0,))))
            P1 = dotg(dcat1(Zr[:, Hh:]), j2r_ref[...], ((1,), (0,)))
            M2 = dotg(dcat1(Zi[:, Hh:]), j2i_ref[...], ((1,), (0,)))
            P3 = dotg(dcat1(Zr[:, Hh:] + Zi[:, Hh:]), j2d_ref[...],
                      ((1,), (0,)))
            Uor = P1 + M2
            Uoi = P3 - P1 + M2
            UerT = jnp.concatenate([Uer, Uer], axis=1)
            UorT = jnp.concatenate([Uor, Uor], axis=1)
            UoiT = jnp.concatenate([Uoi, Uoi], axis=1)
            wxr = wxr_ref[0]
            wxi = wxi_ref[0]
            return UerT + wxr * UorT - wxi * UoiT

        @pl.when(s == 0)
        def _fwd_pair():
            Yr, Yi = t1_complex(x_ref[0, 0], x_ref[0, 1])
            Fzr, Fzi = t2_complex(Yr, Yi)
            fr_scr[0] = Fzr
            fi_scr[0] = Fzi
            pw_scr[...] = Fzr * Fzr + Fzi * Fzi

        @pl.when(s == 1)
        def _fwd_single():
            Yr, Yi = t1_real(x_ref[0, 0])
            Fr, Fi = t2_complex(Yr, Yi)
            fr_scr[1] = Fr
            fi_scr[1] = Fi
            pw = pw_scr[...] + Fr * Fr + Fi * Fi

            total = jnp.maximum(jnp.sum(pw), 1e-12)
            bidx = bidxp_ref[...]
            thr = _P * total

            def loop(_, carry):
                lo, hi = carry
                mid = (lo + hi) // 2
                c = jnp.sum(jnp.where(bidx <= mid.astype(jnp.float32),
                                      pw, 0.0))
                take = c >= thr
                return (jnp.where(take, lo, mid), jnp.where(take, mid, hi))

            lo, hi = jax.lax.fori_loop(
                0, 7, loop, (jnp.int32(-1), jnp.int32(_NBINS)))
            kf = jnp.minimum(hi, _NBINS - 1).astype(jnp.float32)
            lane = jax.lax.broadcasted_iota(
                jnp.int32, (1, 128), 1).astype(jnp.float32)
            d0 = jnp.sum(jnp.where(lane == kf, radii_ref[0], 0.0))
            d0_ref[...] = jnp.broadcast_to(d0, (1, 1, 1))

        @pl.when(s == 2)
        def _inv_pair():
            d0 = d0_ref[...][0, 0, 0]
            mp = (rrp_ref[...] <= d0).astype(jnp.float32)
            Wr = fr_scr[0] * mp
            Wi = fi_scr[0] * mp
            Zr, Zi = i1(Wr, Wi)
            zr, zi = i2_complex(Zr, Zi)
            scale = 1.0 / (H * W)
            IL0 = zr * scale
            IL1 = zi * scale
            il_ref[0, 0] = IL0
            il_ref[0, 1] = IL1
            ih_ref[0, 0] = x_ref[0, 0] - IL0
            ih_ref[0, 1] = x_ref[0, 1] - IL1
            m = (rr_ref[...] <= d0).astype(jnp.float32)
            ml_ref[0, 0] = m
            mh_ref[0, 0] = 1.0 - m

        @pl.when(s == 3)
        def _inv_single():
            d0 = d0_ref[...][0, 0, 0]
            mp = (rrp_ref[...] <= d0).astype(jnp.float32)
            Gr = fr_scr[1] * mp
            Gi = fi_scr[1] * mp
            Zr, Zi = i1(Gr, Gi)
            IL = i2_real(Zr, Zi) * (1.0 / (H * W))
            il_ref[0, 0] = IL
            ih_ref[0, 0] = x_ref[0, 0] - IL

    return body


def kernel(x):
    B, C, H, W = x.shape
    Hh = H // 2
    rr, bidx, radii_pad = _radial_consts(H, W)

    perm = np.concatenate([np.arange(0, H, 2), np.arange(1, H, 2)])
    rrp = rr[perm][:, perm]
    bidxp = bidx[perm][:, perm]

    m = np.arange(Hh, dtype=np.float64) - Hh // 2
    n = np.arange(Hh, dtype=np.float64)
    th = (-2.0 * np.pi / Hh) * np.outer(m, n)
    Er64, Ei64 = np.cos(th), np.sin(th)
    Er = jnp.asarray(Er64, jnp.float32)
    Ei = jnp.asarray(Ei64, jnp.float32)
    Es = jnp.asarray(Er64 + Ei64, jnp.float32)
    Ed = jnp.asarray(Er64 - Ei64, jnp.float32)

    ang1 = (-2.0 * np.pi / H) * np.arange(Hh, dtype=np.float64)
    t1r = jnp.asarray(np.cos(ang1), jnp.float32).reshape(1, 1, Hh)
    t1i = jnp.asarray(np.sin(ang1), jnp.float32).reshape(1, 1, Hh)
    t2r = jnp.asarray(np.broadcast_to(np.cos(ang1)[:, None], (Hh, W)),
                      jnp.float32)
    t2i = jnp.asarray(np.broadcast_to(np.sin(ang1)[:, None], (Hh, W)),
                      jnp.float32)
    ang2 = (2.0 * np.pi / H) * np.arange(H, dtype=np.float64)
    w2r = jnp.asarray(np.broadcast_to(np.cos(ang2)[:, None], (H, W)),
                      jnp.float32)
    w2i = jnp.asarray(np.broadcast_to(np.sin(ang2)[:, None], (H, W)),
                      jnp.float32)
    wxr = jnp.asarray(np.cos(ang2), jnp.float32).reshape(1, 1, H)
    wxi = jnp.asarray(np.sin(ang2), jnp.float32).reshape(1, 1, H)

    def sp(mat):
        return _split_hi_lo(mat)

    Erh, Erl = sp(Er)
    Eih, Eil = sp(Ei)
    Esh, Esl = sp(Es)
    Edh, Edl = sp(Ed)
    cat = jnp.concatenate
    mats = [
        cat([Erh, Erl, Erh], axis=1),
        cat([Eih, Eil, Eih], axis=1),
        cat([Esh, Esl, Esh], axis=1),
        cat([Erh, Erh, Erl], axis=1),
        cat([Eih, Eih, Eil], axis=1),
        cat([Esh, Esh, Esl], axis=1),
        cat([Erh, Erh, Erl], axis=0),
        cat([Eih, Eih, Eil], axis=0),
        cat([Edh, Edh, Edl], axis=0),
        cat([Erh, Erl, Erh], axis=0),
        cat([Eih, Eil, Eih], axis=0),
        cat([Edh, Edl, Edh], axis=0),
    ]

    body = _make_body(C, H, W)

    def xmap(b, s):
        return (b, jnp.where(s < 2, s, s - 2), 0, 0)

    def omap(b, s):
        return (b, jnp.where(s < 2, 0, s - 2), 0, 0)

    c2 = lambda b, s: (0, 0)
    c3 = lambda b, s: (0, 0, 0)

    outs = pl.pallas_call(
        body,
        grid=(B, 4),
        in_specs=[
            pl.BlockSpec((1, 2, H, W), xmap),
            pl.BlockSpec((H, W), c2),
            pl.BlockSpec((H, W), c2),
            pl.BlockSpec((H, W), c2),
            pl.BlockSpec((1, 1, 128), c3),
            pl.BlockSpec((1, 1, Hh), c3),
            pl.BlockSpec((1, 1, Hh), c3),
            pl.BlockSpec((Hh, W), c2),
            pl.BlockSpec((Hh, W), c2),
            pl.BlockSpec((H, W), c2),
            pl.BlockSpec((H, W), c2),
            pl.BlockSpec((1, 1, H), c3),
            pl.BlockSpec((1, 1, H), c3),
        ] + [pl.BlockSpec((Hh, 3 * Hh), c2)] * 6
          + [pl.BlockSpec((3 * Hh, Hh), c2)] * 6,
        out_specs=[
            pl.BlockSpec((1, 2, H, W), omap),
            pl.BlockSpec((1, 2, H, W), omap),
            pl.BlockSpec((1, 1, 1), lambda b, s: (b, 0, 0)),
            pl.BlockSpec((1, 1, H, W), lambda b, s: (b, 0, 0, 0)),
            pl.BlockSpec((1, 1, H, W), lambda b, s: (b, 0, 0, 0)),
        ],
        out_shape=[
            jax.ShapeDtypeStruct((B, C, H, W), jnp.float32),
            jax.ShapeDtypeStruct((B, C, H, W), jnp.float32),
            jax.ShapeDtypeStruct((B, 1, 1), jnp.float32),
            jax.ShapeDtypeStruct((B, 1, H, W), jnp.float32),
            jax.ShapeDtypeStruct((B, 1, H, W), jnp.float32),
        ],
        scratch_shapes=[
            pltpu.VMEM((2, H, W), jnp.float32),
            pltpu.VMEM((2, H, W), jnp.float32),
            pltpu.VMEM((H, W), jnp.float32),
        ],
        compiler_params=pltpu.CompilerParams(
            dimension_semantics=("arbitrary", "arbitrary")),
    )(x, rr, rrp, bidxp, radii_pad,
      t1r, t1i, t2r, t2i, w2r, w2i, wxr, wxi, *mats)

    i_high, i_low, d0, mask_low, mask_high = outs
    return i_high, i_low, d0.reshape(B), mask_low, mask_high


# speedup vs baseline: 1.9196x; 1.0742x over previous
import numpy as np
import jax
import jax.numpy as jnp
from jax.experimental import pallas as pl
from jax.experimental.pallas import tpu as pltpu

_NBINS = 100
_P = 0.95


def _radial_consts(h, w):
    cy, cx = h // 2, w // 2
    yy = jnp.arange(h, dtype=jnp.float32) - cy
    xx = jnp.arange(w, dtype=jnp.float32) - cx
    md = ((h // 2) ** 2 + (w // 2) ** 2) ** 0.5
    yy = yy / (md + 1e-06)
    xx = xx / (md + 1e-06)
    gy, gx = jnp.meshgrid(yy, xx, indexing="ij")
    rr = jnp.sqrt(gx * gx + gy * gy)
    r_max = jnp.max(rr)
    bin_idx = jnp.floor(rr / r_max * _NBINS).astype(jnp.int32)
    bidx = jnp.where(bin_idx < _NBINS,
                     jnp.clip(bin_idx, 0, _NBINS - 1), 127).astype(jnp.float32)
    edges = jnp.linspace(0.0, r_max, _NBINS + 1)
    radii = (edges[:-1] + edges[1:]) * 0.5
    radii_pad = jnp.concatenate(
        [radii, jnp.full((28,), radii[-1], jnp.float32)]).reshape(1, 1, 128)
    return rr, bidx, radii_pad


def _split_hi_lo(a):
    hi = a.astype(jnp.bfloat16)
    lo = (a - hi.astype(jnp.float32)).astype(jnp.bfloat16)
    return hi, lo


def _make_body(C, H, W):
    Hh = H // 2

    def dotg(a, b, dims):
        return jax.lax.dot_general(a, b, (dims, ((), ())),
                                   preferred_element_type=jnp.float32)

    def dcat1(v):
        vh, vl = _split_hi_lo(v)
        return jnp.concatenate([vh, vh, vl], axis=1)

    def dcat0(v):
        vh, vl = _split_hi_lo(v)
        return jnp.concatenate([vh, vl, vh], axis=0)

    def body(x_ref, rr_ref, rrp_ref, bidxp_ref, radii_ref,
             e1r_ref, e1i_ref, e1s_ref, e1tr_ref, e1ti_ref, e1ts_ref,
             e2r_ref, e2i_ref, e2s_ref, e2tr_ref, e2ti_ref, e2ts_ref,
             j1r_ref, j1i_ref, j1d_ref, j1tr_ref, j1ti_ref, j1ts_ref,
             j2r_ref, j2i_ref, j2d_ref, j2tr_ref, j2ti_ref, j2ts_ref,
             ih_ref, il_ref, d0_ref, ml_ref, mh_ref,
             fr_scr, fi_scr, pw_scr):
        s = pl.program_id(1)

        def t1_complex(Xr, Xi):
            Arr = Xr[:, :Hh] + Xr[:, Hh:]
            Aii = Xi[:, :Hh] + Xi[:, Hh:]
            Brr = Xr[:, :Hh] - Xr[:, Hh:]
            Bii = Xi[:, :Hh] - Xi[:, Hh:]
            P1 = dotg(dcat1(Arr), e1r_ref[...], ((1,), (1,)))
            P2 = dotg(dcat1(Aii), e1i_ref[...], ((1,), (1,)))
            P3 = dotg(dcat1(Arr + Aii), e1s_ref[...], ((1,), (1,)))
            Q1 = dotg(dcat1(Brr), e1tr_ref[...], ((1,), (1,)))
            Q2 = dotg(dcat1(Bii), e1ti_ref[...], ((1,), (1,)))
            Q3 = dotg(dcat1(Brr + Bii), e1ts_ref[...], ((1,), (1,)))
            Yr = jnp.concatenate([P1 - P2, Q1 - Q2], axis=1)
            Yi = jnp.concatenate([P3 - P1 - P2, Q3 - Q1 - Q2], axis=1)
            return Yr, Yi

        def t1_real(X):
            A = X[:, :Hh] + X[:, Hh:]
            B = X[:, :Hh] - X[:, Hh:]
            Ac = dcat1(A)
            Bc = dcat1(B)
            Yr = jnp.concatenate(
                [dotg(Ac, e1r_ref[...], ((1,), (1,))),
                 dotg(Bc, e1tr_ref[...], ((1,), (1,)))], axis=1)
            Yi = jnp.concatenate(
                [dotg(Ac, e1i_ref[...], ((1,), (1,))),
                 dotg(Bc, e1ti_ref[...], ((1,), (1,)))], axis=1)
            return Yr, Yi

        def t2_complex(Yr, Yi):
            A2r = Yr[:Hh, :] + Yr[Hh:, :]
            A2i = Yi[:Hh, :] + Yi[Hh:, :]
            B2r = Yr[:Hh, :] - Yr[Hh:, :]
            B2i = Yi[:Hh, :] - Yi[Hh:, :]
            P1 = dotg(e2r_ref[...], dcat0(A2r), ((1,), (0,)))
            P2 = dotg(e2i_ref[...], dcat0(A2i), ((1,), (0,)))
            P3 = dotg(e2s_ref[...], dcat0(A2r + A2i), ((1,), (0,)))
            Q1 = dotg(e2tr_ref[...], dcat0(B2r), ((1,), (0,)))
            Q2 = dotg(e2ti_ref[...], dcat0(B2i), ((1,), (0,)))
            Q3 = dotg(e2ts_ref[...], dcat0(B2r + B2i), ((1,), (0,)))
            Fr = jnp.concatenate([P1 - P2, Q1 - Q2], axis=0)
            Fi = jnp.concatenate([P3 - P1 - P2, Q3 - Q1 - Q2], axis=0)
            return Fr, Fi

        def i1(Wr, Wi):
            evr, evi = Wr[:Hh, :], Wi[:Hh, :]
            odr, odi = Wr[Hh:, :], Wi[Hh:, :]
            P1 = dotg(j1r_ref[...], dcat0(evr), ((0,), (0,)))
            M2 = dotg(j1i_ref[...], dcat0(evi), ((0,), (0,)))
            P3 = dotg(j1d_ref[...], dcat0(evr + evi), ((0,), (0,)))
            Ter = P1 + M2
            Tei = P3 - P1 + M2
            Q1 = dotg(j1tr_ref[...], dcat0(odr), ((0,), (0,)))
            Q2 = dotg(j1ti_ref[...], dcat0(odi), ((0,), (0,)))
            Q3 = dotg(j1ts_ref[...], dcat0(odr + odi), ((0,), (0,)))
            Twr = Q1 - Q2
            Twi = Q3 - Q1 - Q2
            Zr = jnp.concatenate([Ter + Twr, Ter - Twr], axis=0)
            Zi = jnp.concatenate([Tei + Twi, Tei - Twi], axis=0)
            return Zr, Zi

        def i2_complex(Zr, Zi):
            P1 = dotg(dcat1(Zr[:, :Hh]), j2r_ref[...], ((1,), (0,)))
            M2 = dotg(dcat1(Zi[:, :Hh]), j2i_ref[...], ((1,), (0,)))
            P3 = dotg(dcat1(Zr[:, :Hh] + Zi[:, :Hh]), j2d_ref[...],
                      ((1,), (0,)))
            Uer = P1 + M2
            Uei = P3 - P1 + M2
            Q1 = dotg(dcat1(Zr[:, Hh:]), j2tr_ref[...], ((1,), (0,)))
            Q2 = dotg(dcat1(Zi[:, Hh:]), j2ti_ref[...], ((1,), (0,)))
            Q3 = dotg(dcat1(Zr[:, Hh:] + Zi[:, Hh:]), j2ts_ref[...],
                      ((1,), (0,)))
            Utr = Q1 - Q2
            Uti = Q3 - Q1 - Q2
            zr = jnp.concatenate([Uer + Utr, Uer - Utr], axis=1)
            zi = jnp.concatenate([Uei + Uti, Uei - Uti], axis=1)
            return zr, zi

        def i2_real(Zr, Zi):
            UeR = (dotg(dcat1(Zr[:, :Hh]), j2r_ref[...], ((1,), (0,))) +
                   dotg(dcat1(Zi[:, :Hh]), j2i_ref[...], ((1,), (0,))))
            UtR = (dotg(dcat1(Zr[:, Hh:]), j2tr_ref[...], ((1,), (0,))) -
                   dotg(dcat1(Zi[:, Hh:]), j2ti_ref[...], ((1,), (0,))))
            return jnp.concatenate([UeR + UtR, UeR - UtR], axis=1)

        @pl.when(s == 0)
        def _fwd_pair():
            Yr, Yi = t1_complex(x_ref[0, 0], x_ref[0, 1])
            Fzr, Fzi = t2_complex(Yr, Yi)
            fr_scr[0] = Fzr
            fi_scr[0] = Fzi
            pw_scr[...] = Fzr * Fzr + Fzi * Fzi

        @pl.when(s == 1)
        def _fwd_single():
            Yr, Yi = t1_real(x_ref[0, 0])
            Fr, Fi = t2_complex(Yr, Yi)
            fr_scr[1] = Fr
            fi_scr[1] = Fi
            pw = pw_scr[...] + Fr * Fr + Fi * Fi

            total = jnp.maximum(jnp.sum(pw), 1e-12)
            bidx = bidxp_ref[...]
            thr = _P * total

            def loop(_, carry):
                lo, hi = carry
                mid = (lo + hi) // 2
                c = jnp.sum(jnp.where(bidx <= mid.astype(jnp.float32),
                                      pw, 0.0))
                take = c >= thr
                return (jnp.where(take, lo, mid), jnp.where(take, mid, hi))

            lo, hi = jax.lax.fori_loop(
                0, 7, loop, (jnp.int32(-1), jnp.int32(_NBINS)))
            kf = jnp.minimum(hi, _NBINS - 1).astype(jnp.float32)
            lane = jax.lax.broadcasted_iota(
                jnp.int32, (1, 128), 1).astype(jnp.float32)
            d0 = jnp.sum(jnp.where(lane == kf, radii_ref[0], 0.0))
            d0_ref[...] = jnp.broadcast_to(d0, (1, 1, 1))

        @pl.when(s == 2)
        def _inv_pair():
            d0 = d0_ref[...][0, 0, 0]
            mp = (rrp_ref[...] <= d0).astype(jnp.float32)
            Zr, Zi = i1(fr_scr[0] * mp, fi_scr[0] * mp)
            zr, zi = i2_complex(Zr, Zi)
            scale = 1.0 / (H * W)
            IL0 = zr * scale
            IL1 = zi * scale
            il_ref[0, 0] = IL0
            il_ref[0, 1] = IL1
            ih_ref[0, 0] = x_ref[0, 0] - IL0
            ih_ref[0, 1] = x_ref[0, 1] - IL1
            m = (rr_ref[...] <= d0).astype(jnp.float32)
            ml_ref[0, 0] = m
            mh_ref[0, 0] = 1.0 - m

        @pl.when(s == 3)
        def _inv_single():
            d0 = d0_ref[...][0, 0, 0]
            mp = (rrp_ref[...] <= d0).astype(jnp.float32)
            Zr, Zi = i1(fr_scr[1] * mp, fi_scr[1] * mp)
            IL = i2_real(Zr, Zi) * (1.0 / (H * W))
            il_ref[0, 0] = IL
            ih_ref[0, 0] = x_ref[0, 0] - IL

    return body


def kernel(x):
    B, C, H, W = x.shape
    Hh = H // 2
    rr, bidx, radii_pad = _radial_consts(H, W)

    perm = np.concatenate([np.arange(0, H, 2), np.arange(1, H, 2)])
    rrp = rr[perm][:, perm]
    bidxp = bidx[perm][:, perm]

    m = np.arange(Hh, dtype=np.float64) - Hh // 2
    n = np.arange(Hh, dtype=np.float64)
    th = (-2.0 * np.pi / Hh) * np.outer(m, n)
    Er64, Ei64 = np.cos(th), np.sin(th)
    c = np.cos(2.0 * np.pi * n / H)
    sn = np.sin(2.0 * np.pi * n / H)
    Etwr64 = c * Er64 + sn * Ei64
    Etwi64 = c * Ei64 - sn * Er64

    def f32(a):
        return jnp.asarray(a, jnp.float32)

    Er, Ei = f32(Er64), f32(Ei64)
    Es, Ed = f32(Er64 + Ei64), f32(Er64 - Ei64)
    Etr, Eti = f32(Etwr64), f32(Etwi64)
    Ets = f32(Etwr64 + Etwi64)
    Mr, Mi = Etr, f32(-Etwi64)
    Ms = f32(Etwr64 - Etwi64)

    def sp(mat):
        return _split_hi_lo(mat)

    cat = jnp.concatenate

    def c_dl1(mat):
        h_, l_ = sp(mat)
        return cat([h_, l_, h_], axis=1)

    def c_cl1(mat):
        h_, l_ = sp(mat)
        return cat([h_, h_, l_], axis=1)

    def c_cl0(mat):
        h_, l_ = sp(mat)
        return cat([h_, h_, l_], axis=0)

    def c_dl0(mat):
        h_, l_ = sp(mat)
        return cat([h_, l_, h_], axis=0)

    mats = [
        c_dl1(Er), c_dl1(Ei), c_dl1(Es), c_dl1(Etr), c_dl1(Eti), c_dl1(Ets),
        c_cl1(Er), c_cl1(Ei), c_cl1(Es), c_cl1(Etr), c_cl1(Eti), c_cl1(Ets),
        c_cl0(Er), c_cl0(Ei), c_cl0(Ed), c_cl0(Mr), c_cl0(Mi), c_cl0(Ms),
        c_dl0(Er), c_dl0(Ei), c_dl0(Ed), c_dl0(Mr), c_dl0(Mi), c_dl0(Ms),
    ]

    body = _make_body(C, H, W)

    def xmap(b, s):
        return (b, jnp.where(s < 2, s, s - 2), 0, 0)

    def omap(b, s):
        return (b, jnp.where(s < 2, 0, s - 2), 0, 0)

    c2 = lambda b, s: (0, 0)
    c3 = lambda b, s: (0, 0, 0)

    outs = pl.pallas_call(
        body,
        grid=(B, 4),
        in_specs=[
            pl.BlockSpec((1, 2, H, W), xmap),
            pl.BlockSpec((H, W), c2),
            pl.BlockSpec((H, W), c2),
            pl.BlockSpec((H, W), c2),
            pl.BlockSpec((1, 1, 128), c3),
        ] + [pl.BlockSpec((Hh, 3 * Hh), c2)] * 12
          + [pl.BlockSpec((3 * Hh, Hh), c2)] * 12,
        out_specs=[
            pl.BlockSpec((1, 2, H, W), omap),
            pl.BlockSpec((1, 2, H, W), omap),
            pl.BlockSpec((1, 1, 1), lambda b, s: (b, 0, 0)),
            pl.BlockSpec((1, 1, H, W), lambda b, s: (b, 0, 0, 0)),
            pl.BlockSpec((1, 1, H, W), lambda b, s: (b, 0, 0, 0)),
        ],
        out_shape=[
            jax.ShapeDtypeStruct((B, C, H, W), jnp.float32),
            jax.ShapeDtypeStruct((B, C, H, W), jnp.float32),
            jax.ShapeDtypeStruct((B, 1, 1), jnp.float32),
            jax.ShapeDtypeStruct((B, 1, H, W), jnp.float32),
            jax.ShapeDtypeStruct((B, 1, H, W), jnp.float32),
        ],
        scratch_shapes=[
            pltpu.VMEM((2, H, W), jnp.float32),
            pltpu.VMEM((2, H, W), jnp.float32),
            pltpu.VMEM((H, W), jnp.float32),
        ],
        compiler_params=pltpu.CompilerParams(
            dimension_semantics=("arbitrary", "arbitrary")),
    )(x, rr, rrp, bidxp, radii_pad, *mats)

    i_high, i_low, d0, mask_low, mask_high = outs
    return i_high, i_low, d0.reshape(B), mask_low, mask_high
